# Initial kernel scaffold; baseline (speedup 1.0000x reference)
import jax
import jax.numpy as jnp
from jax.experimental import pallas as pl


def kernel(stack, data, cursors, stack_mask, mask, W_R, b_R, W_S1, b_S1, W_S2, b_S2):
    raise NotImplementedError("write your pallas kernel here")


# speedup vs baseline: 1.0593x; 1.0593x over previous
import functools

import jax
import jax.numpy as jnp
from jax import lax
from jax.experimental import pallas as pl
from jax.experimental.pallas import tpu as pltpu
from jax.experimental.pallas import tpu_sc as plsc


def _idx_body(cur_ref, sm_ref, sp_ref, spp_ref, ic_ref):
    B, L = cur_ref.shape
    cl = jnp.sum(cur_ref[...], axis=1, keepdims=True).astype(jnp.int32)
    sk = jnp.sum(sm_ref[...], axis=1, keepdims=True).astype(jnp.int32)
    base = lax.broadcasted_iota(jnp.int32, (B, 1), 0) * L
    sp_ref[...] = base + jnp.clip(sk - 2, 0, L - 1)
    spp_ref[...] = base + jnp.clip(sk - 3, 0, L - 1)
    ic_ref[...] = base + jnp.clip(cl - 1, 0, L - 1)


def _make_gather(B, D):
    info = plsc.get_sparse_core_info()
    NC, NS = info.num_cores, info.num_subcores
    bpw = B // (NC * NS)
    mesh = plsc.VectorSubcoreMesh(core_axis_name="c", subcore_axis_name="s")

    @functools.partial(
        pl.kernel, mesh=mesh,
        out_type=[jax.ShapeDtypeStruct((B, D), jnp.float32)] * 3,
        scratch_types=(
            [pltpu.VMEM((bpw,), jnp.int32)] * 3
            + [pltpu.VMEM((bpw, D), jnp.float32)] * 3
            + [pltpu.SemaphoreType.DMA] * 3
        ),
    )
    def gather(stack_hbm, data_hbm, spi_hbm, sppi_hbm, ici_hbm,
               sp_out, spp_out, ic_out,
               iv0, iv1, iv2, r0, r1, r2, s0, s1, s2):
        wid = lax.axis_index("s") * NC + lax.axis_index("c")
        base = wid * bpw
        pltpu.sync_copy(spi_hbm.at[pl.ds(base, bpw)], iv0)
        pltpu.sync_copy(sppi_hbm.at[pl.ds(base, bpw)], iv1)
        pltpu.sync_copy(ici_hbm.at[pl.ds(base, bpw)], iv2)
        c0 = pltpu.async_copy(stack_hbm.at[iv0], r0, s0)
        c1 = pltpu.async_copy(stack_hbm.at[iv1], r1, s1)
        c2 = pltpu.async_copy(data_hbm.at[iv2], r2, s2)
        c0.wait()
        c1.wait()
        c2.wait()
        pltpu.sync_copy(r0, sp_out.at[pl.ds(base, bpw)])
        pltpu.sync_copy(r1, spp_out.at[pl.ds(base, bpw)])
        pltpu.sync_copy(r2, ic_out.at[pl.ds(base, bpw)])

    return gather


def _compute_body(sp_ref, spp_ref, ic_ref, cur_ref, sm_ref, mask_ref,
                  wr_ref, br_ref, ws1_ref, bs1_ref, ws2_ref, bs2_ref,
                  act_ref, red_ref, ice_ref):
    B, D = sp_ref.shape
    H = D // 2
    sk = jnp.sum(sm_ref[...], axis=1, keepdims=True)
    cl = jnp.sum(cur_ref[...], axis=1, keepdims=True)
    sl = jnp.sum(mask_ref[...], axis=1, keepdims=True)
    skb = lax.broadcast_in_dim(sk, (B, D), (0, 1))
    clb = lax.broadcast_in_dim(cl, (B, D), (0, 1))
    slb = lax.broadcast_in_dim(sl, (B, D), (0, 1))
    sp = jnp.where(skb >= 2.0, sp_ref[...], 0.0)
    spp = jnp.where(skb >= 3.0, spp_ref[...], 0.0)
    ic = jnp.where(clb >= 1.0, ic_ref[...], 0.0)
    h = jnp.concatenate([sp[:, H:], spp[:, H:]], axis=1)
    q = jnp.dot(h, wr_ref[...], preferred_element_type=jnp.float32) + br_ref[...]
    q1 = jax.nn.sigmoid(q[:, : 4 * H])
    q2 = jnp.tanh(q[:, 4 * H :])
    c = (q1[:, H : 2 * H] * sp[:, :H]
         + q1[:, 2 * H : 3 * H] * spp[:, :H]
         + q1[:, :H] * q2)
    hh = q1[:, 3 * H :] * c
    red = jnp.concatenate([c, hh], axis=1)
    s_in = jnp.concatenate([sp[:, H:], spp[:, H:], ic[:, H:]], axis=1)
    s = jnp.maximum(
        jnp.dot(s_in, ws1_ref[...], preferred_element_type=jnp.float32) + bs1_ref[...],
        0.0)
    logits = jnp.dot(s, ws2_ref[...], preferred_element_type=jnp.float32) + bs2_ref[...]
    pol = jnp.exp(logits)
    p0 = lax.broadcast_in_dim(pol[:, 0:1], (B, D), (0, 1))
    p1 = lax.broadcast_in_dim(pol[:, 1:2], (B, D), (0, 1))
    shift = p0 >= p1
    input_is_empty = (slb - clb) == -1.0
    stack_is_empty = skb <= 1.0
    shift = (shift & (~input_is_empty)) | stack_is_empty
    act_ref[...] = jnp.where(shift, 1.0, 0.0)
    red_ref[...] = red
    ice_ref[...] = ic


def _scatter_body(stack_ref, sm_ref, act_ref, red_ref, ic_ref, out_ref):
    Bblk, L, D = stack_ref.shape
    sk = jnp.sum(sm_ref[...], axis=1, keepdims=True).astype(jnp.int32)
    shift = act_ref[...] > 0.5
    pos = lax.broadcasted_iota(jnp.int32, (Bblk, L, 1), 1)
    m_shift = (pos == jnp.minimum(sk, L - 1)) & shift
    m_zero = (pos == sk - 2) & (~shift)
    m_red = (pos == sk - 3) & (~shift)
    full = (Bblk, L, D)
    out = jnp.where(lax.broadcast_in_dim(m_zero, full, (0, 1, 2)), 0.0,
                    stack_ref[...])
    out = jnp.where(lax.broadcast_in_dim(m_red, full, (0, 1, 2)),
                    lax.broadcast_in_dim(red_ref[...], full, (0, 1, 2)), out)
    out = jnp.where(lax.broadcast_in_dim(m_shift, full, (0, 1, 2)),
                    lax.broadcast_in_dim(ic_ref[...], full, (0, 1, 2)), out)
    out_ref[...] = out


def kernel(stack, data, cursors, stack_mask, mask, W_R, b_R, W_S1, b_S1, W_S2, b_S2):
    B, L, D = stack.shape
    H = D // 2
    RL = W_S1.shape[1]
    sp_i, spp_i, ic_i = pl.pallas_call(
        _idx_body,
        out_shape=[jax.ShapeDtypeStruct((B, 1), jnp.int32)] * 3,
    )(cursors, stack_mask)
    gather = _make_gather(B, D)
    sp_rows, spp_rows, ic_rows = gather(
        stack.reshape(B * L, D), data.reshape(B * L, D),
        sp_i.reshape(B), spp_i.reshape(B), ic_i.reshape(B))
    ws2p = jnp.zeros((RL, 128), jnp.float32).at[:, : W_S2.shape[1]].set(W_S2)
    bs2p = jnp.zeros((1, 128), jnp.float32).at[:, : b_S2.shape[0]].set(b_S2[None, :])
    act, red, ic_eff = pl.pallas_call(
        _compute_body,
        out_shape=[jax.ShapeDtypeStruct((B, D), jnp.float32)] * 3,
    )(sp_rows, spp_rows, ic_rows, cursors, stack_mask, mask,
      W_R, b_R[None, :], W_S1, b_S1[None, :], ws2p, bs2p)
    Bblk = 64
    return pl.pallas_call(
        _scatter_body,
        grid=(B // Bblk,),
        in_specs=[
            pl.BlockSpec((Bblk, L, D), lambda i: (i, 0, 0)),
            pl.BlockSpec((Bblk, L, 1), lambda i: (i, 0, 0)),
            pl.BlockSpec((Bblk, 1, 1), lambda i: (i, 0, 0)),
            pl.BlockSpec((Bblk, 1, D), lambda i: (i, 0, 0)),
            pl.BlockSpec((Bblk, 1, D), lambda i: (i, 0, 0)),
        ],
        out_specs=pl.BlockSpec((Bblk, L, D), lambda i: (i, 0, 0)),
        out_shape=jax.ShapeDtypeStruct((B, L, D), jnp.float32),
        compiler_params=pltpu.CompilerParams(dimension_semantics=("arbitrary",)),
    )(stack, stack_mask.reshape(B, L, 1), act[:, :1].reshape(B, 1, 1),
      red.reshape(B, 1, D), ic_eff.reshape(B, 1, D))


# speedup vs baseline: 1.8537x; 1.7499x over previous
import functools

import jax
import jax.numpy as jnp
from jax import lax
from jax.experimental import pallas as pl
from jax.experimental.pallas import tpu as pltpu
from jax.experimental.pallas import tpu_sc as plsc


def _idx_body(cur_ref, sm_ref, sp_ref, spp_ref, ic_ref):
    B, L = cur_ref.shape
    cl = jnp.sum(cur_ref[...], axis=1, keepdims=True).astype(jnp.int32)
    sk = jnp.sum(sm_ref[...], axis=1, keepdims=True).astype(jnp.int32)
    base = lax.broadcasted_iota(jnp.int32, (B, 1), 0) * L
    sp_ref[...] = base + jnp.clip(sk - 2, 0, L - 1)
    spp_ref[...] = base + jnp.clip(sk - 3, 0, L - 1)
    ic_ref[...] = base + jnp.clip(cl - 1, 0, L - 1)


def _make_gather(B, D):
    info = plsc.get_sparse_core_info()
    NC, NS = info.num_cores, info.num_subcores
    bpw = B // (NC * NS)
    mesh = plsc.VectorSubcoreMesh(core_axis_name="c", subcore_axis_name="s")

    @functools.partial(
        pl.kernel, mesh=mesh,
        out_type=[jax.ShapeDtypeStruct((B, D), jnp.float32)] * 3,
        scratch_types=(
            [pltpu.VMEM((bpw,), jnp.int32)] * 3
            + [pltpu.VMEM((bpw, D), jnp.float32)] * 3
            + [pltpu.SemaphoreType.DMA] * 3
        ),
    )
    def gather(stack_hbm, data_hbm, spi_hbm, sppi_hbm, ici_hbm,
               sp_out, spp_out, ic_out,
               iv0, iv1, iv2, r0, r1, r2, s0, s1, s2):
        wid = lax.axis_index("s") * NC + lax.axis_index("c")
        base = wid * bpw
        pltpu.sync_copy(spi_hbm.at[pl.ds(base, bpw)], iv0)
        pltpu.sync_copy(sppi_hbm.at[pl.ds(base, bpw)], iv1)
        pltpu.sync_copy(ici_hbm.at[pl.ds(base, bpw)], iv2)
        c0 = pltpu.async_copy(stack_hbm.at[iv0], r0, s0)
        c1 = pltpu.async_copy(stack_hbm.at[iv1], r1, s1)
        c2 = pltpu.async_copy(data_hbm.at[iv2], r2, s2)
        c0.wait()
        c1.wait()
        c2.wait()
        pltpu.sync_copy(r0, sp_out.at[pl.ds(base, bpw)])
        pltpu.sync_copy(r1, spp_out.at[pl.ds(base, bpw)])
        pltpu.sync_copy(r2, ic_out.at[pl.ds(base, bpw)])

    return gather


def _compute_body(sp_ref, spp_ref, ic_ref, cur_ref, sm_ref, mask_ref,
                  wr_ref, br_ref, ws1_ref, bs1_ref, ws2_ref, bs2_ref,
                  r1_ref, r2_ref, v1_ref, v2_ref):
    B, D = sp_ref.shape
    H = D // 2
    sk = jnp.sum(sm_ref[...], axis=1, keepdims=True)
    cl = jnp.sum(cur_ref[...], axis=1, keepdims=True)
    sl = jnp.sum(mask_ref[...], axis=1, keepdims=True)
    skb = lax.broadcast_in_dim(sk, (B, D), (0, 1))
    clb = lax.broadcast_in_dim(cl, (B, D), (0, 1))
    slb = lax.broadcast_in_dim(sl, (B, D), (0, 1))
    sp = jnp.where(skb >= 2.0, sp_ref[...], 0.0)
    spp = jnp.where(skb >= 3.0, spp_ref[...], 0.0)
    ic = jnp.where(clb >= 1.0, ic_ref[...], 0.0)
    h = jnp.concatenate([sp[:, H:], spp[:, H:]], axis=1)
    q = jnp.dot(h, wr_ref[...], preferred_element_type=jnp.float32) + br_ref[...]
    q1 = jax.nn.sigmoid(q[:, : 4 * H])
    q2 = jnp.tanh(q[:, 4 * H :])
    c = (q1[:, H : 2 * H] * sp[:, :H]
         + q1[:, 2 * H : 3 * H] * spp[:, :H]
         + q1[:, :H] * q2)
    hh = q1[:, 3 * H :] * c
    red = jnp.concatenate([c, hh], axis=1)
    s_in = jnp.concatenate([sp[:, H:], spp[:, H:], ic[:, H:]], axis=1)
    s = jnp.maximum(
        jnp.dot(s_in, ws1_ref[...], preferred_element_type=jnp.float32) + bs1_ref[...],
        0.0)
    logits = jnp.dot(s, ws2_ref[...], preferred_element_type=jnp.float32) + bs2_ref[...]
    pol = jnp.exp(logits)
    p0 = lax.broadcast_in_dim(pol[:, 0:1], (B, D), (0, 1))
    p1 = lax.broadcast_in_dim(pol[:, 1:2], (B, D), (0, 1))
    shift = p0 >= p1
    input_is_empty = (slb - clb) == -1.0
    stack_is_empty = skb <= 1.0
    shift = (shift & (~input_is_empty)) | stack_is_empty
    ski = skb.astype(jnp.int32)
    L1 = jnp.int32(mask_ref.shape[1] - 1)
    r1w = jnp.where(shift, jnp.minimum(ski, L1), ski - 2)
    r2w = jnp.where(shift, jnp.minimum(ski, L1),
                    jnp.where(skb >= 3.0, ski - 3, ski - 2))
    r1_ref[...] = r1w[:, :1]
    r2_ref[...] = r2w[:, :1]
    v1_ref[...] = jnp.where(shift, ic, 0.0)
    v2_ref[...] = jnp.where(shift, ic, jnp.where(skb >= 3.0, red, 0.0))


def _scatter_body(r1_ref, r2_ref, stack_ref, v1_ref, v2_ref, out_ref):
    Bblk, L, D = stack_ref.shape
    out_ref[...] = stack_ref[...]
    i = pl.program_id(0)

    def body(b, carry):
        g = i * Bblk + b
        rr1 = r1_ref[g]
        rr2 = r2_ref[g]
        out_ref[pl.ds(b, 1), pl.ds(rr1, 1), :] = v1_ref[pl.ds(b, 1)]
        out_ref[pl.ds(b, 1), pl.ds(rr2, 1), :] = v2_ref[pl.ds(b, 1)]
        return carry

    lax.fori_loop(0, Bblk, body, 0)


def kernel(stack, data, cursors, stack_mask, mask, W_R, b_R, W_S1, b_S1, W_S2, b_S2):
    B, L, D = stack.shape
    H = D // 2
    RL = W_S1.shape[1]
    sp_i, spp_i, ic_i = pl.pallas_call(
        _idx_body,
        out_shape=[jax.ShapeDtypeStruct((B, 1), jnp.int32)] * 3,
    )(cursors, stack_mask)
    gather = _make_gather(B, D)
    sp_rows, spp_rows, ic_rows = gather(
        stack.reshape(B * L, D), data.reshape(B * L, D),
        sp_i.reshape(B), spp_i.reshape(B), ic_i.reshape(B))
    ws2p = jnp.zeros((RL, 128), jnp.float32).at[:, : W_S2.shape[1]].set(W_S2)
    bs2p = jnp.zeros((1, 128), jnp.float32).at[:, : b_S2.shape[0]].set(b_S2[None, :])
    r1, r2, v1, v2 = pl.pallas_call(
        _compute_body,
        out_shape=[
            jax.ShapeDtypeStruct((B, 1), jnp.int32),
            jax.ShapeDtypeStruct((B, 1), jnp.int32),
            jax.ShapeDtypeStruct((B, D), jnp.float32),
            jax.ShapeDtypeStruct((B, D), jnp.float32),
        ],
    )(sp_rows, spp_rows, ic_rows, cursors, stack_mask, mask,
      W_R, b_R[None, :], W_S1, b_S1[None, :], ws2p, bs2p)
    Bblk = 128
    return pl.pallas_call(
        _scatter_body,
        grid_spec=pltpu.PrefetchScalarGridSpec(
            num_scalar_prefetch=2,
            grid=(B // Bblk,),
            in_specs=[
                pl.BlockSpec((Bblk, L, D), lambda i, r1s, r2s: (i, 0, 0)),
                pl.BlockSpec((Bblk, 1, D), lambda i, r1s, r2s: (i, 0, 0)),
                pl.BlockSpec((Bblk, 1, D), lambda i, r1s, r2s: (i, 0, 0)),
            ],
            out_specs=pl.BlockSpec((Bblk, L, D), lambda i, r1s, r2s: (i, 0, 0)),
        ),
        out_shape=jax.ShapeDtypeStruct((B, L, D), jnp.float32),
        compiler_params=pltpu.CompilerParams(dimension_semantics=("arbitrary",)),
    )(r1.reshape(B), r2.reshape(B), stack, v1.reshape(B, 1, D), v2.reshape(B, 1, D))


# speedup vs baseline: 1.8984x; 1.0241x over previous
import functools

import jax
import jax.numpy as jnp
from jax import lax
from jax.experimental import pallas as pl
from jax.experimental.pallas import tpu as pltpu
from jax.experimental.pallas import tpu_sc as plsc


def _idx_body(cur_ref, sm_ref, sp_ref, spp_ref, ic_ref):
    B, L = cur_ref.shape
    cl = jnp.sum(cur_ref[...], axis=1, keepdims=True).astype(jnp.int32)
    sk = jnp.sum(sm_ref[...], axis=1, keepdims=True).astype(jnp.int32)
    base = lax.broadcasted_iota(jnp.int32, (B, 1), 0) * L
    sp_ref[...] = base + jnp.clip(sk - 2, 0, L - 1)
    spp_ref[...] = base + jnp.clip(sk - 3, 0, L - 1)
    ic_ref[...] = base + jnp.clip(cl - 1, 0, L - 1)


def _make_gather(B, D):
    info = plsc.get_sparse_core_info()
    NC, NS = info.num_cores, info.num_subcores
    bpw = B // (NC * NS)
    mesh = plsc.VectorSubcoreMesh(core_axis_name="c", subcore_axis_name="s")

    @functools.partial(
        pl.kernel, mesh=mesh,
        out_type=[jax.ShapeDtypeStruct((B, D), jnp.float32)] * 3,
        scratch_types=(
            [pltpu.VMEM((bpw,), jnp.int32)] * 3
            + [pltpu.VMEM((bpw, D), jnp.float32)] * 3
            + [pltpu.SemaphoreType.DMA] * 3
        ),
    )
    def gather(stack_hbm, data_hbm, spi_hbm, sppi_hbm, ici_hbm,
               sp_out, spp_out, ic_out,
               iv0, iv1, iv2, r0, r1, r2, s0, s1, s2):
        wid = lax.axis_index("s") * NC + lax.axis_index("c")
        base = wid * bpw
        pltpu.sync_copy(spi_hbm.at[pl.ds(base, bpw)], iv0)
        pltpu.sync_copy(sppi_hbm.at[pl.ds(base, bpw)], iv1)
        pltpu.sync_copy(ici_hbm.at[pl.ds(base, bpw)], iv2)
        c0 = pltpu.async_copy(stack_hbm.at[iv0], r0, s0)
        c1 = pltpu.async_copy(stack_hbm.at[iv1], r1, s1)
        c2 = pltpu.async_copy(data_hbm.at[iv2], r2, s2)
        c0.wait()
        c1.wait()
        c2.wait()
        pltpu.sync_copy(r0, sp_out.at[pl.ds(base, bpw)])
        pltpu.sync_copy(r1, spp_out.at[pl.ds(base, bpw)])
        pltpu.sync_copy(r2, ic_out.at[pl.ds(base, bpw)])

    return gather


def _main_body(sp_ref, spp_ref, ic_ref, cur_ref, sm_ref, mask_ref,
               wr_ref, br_ref, ws1_ref, bs1_ref, ws2_ref, bs2_ref, stack_ref,
               out_ref, r1v, r2v, v1s, v2s):
    Bblk, L, D = stack_ref.shape
    H = D // 2
    sk = jnp.sum(sm_ref[...], axis=1, keepdims=True)
    cl = jnp.sum(cur_ref[...], axis=1, keepdims=True)
    sl = jnp.sum(mask_ref[...], axis=1, keepdims=True)
    skb = lax.broadcast_in_dim(sk, (Bblk, D), (0, 1))
    clb = lax.broadcast_in_dim(cl, (Bblk, D), (0, 1))
    slb = lax.broadcast_in_dim(sl, (Bblk, D), (0, 1))
    sp = jnp.where(skb >= 2.0, sp_ref[...], 0.0)
    spp = jnp.where(skb >= 3.0, spp_ref[...], 0.0)
    ic = jnp.where(clb >= 1.0, ic_ref[...], 0.0)
    h = jnp.concatenate([sp[:, H:], spp[:, H:]], axis=1)
    q = jnp.dot(h, wr_ref[...], preferred_element_type=jnp.float32) + br_ref[...]
    q1 = jax.nn.sigmoid(q[:, : 4 * H])
    q2 = jnp.tanh(q[:, 4 * H :])
    c = (q1[:, H : 2 * H] * sp[:, :H]
         + q1[:, 2 * H : 3 * H] * spp[:, :H]
         + q1[:, :H] * q2)
    hh = q1[:, 3 * H :] * c
    red = jnp.concatenate([c, hh], axis=1)
    s_in = jnp.concatenate([sp[:, H:], spp[:, H:], ic[:, H:]], axis=1)
    s = jnp.maximum(
        jnp.dot(s_in, ws1_ref[...], preferred_element_type=jnp.float32) + bs1_ref[...],
        0.0)
    logits = jnp.dot(s, ws2_ref[...], preferred_element_type=jnp.float32) + bs2_ref[...]
    pol = jnp.exp(logits)
    p0 = lax.broadcast_in_dim(pol[:, 0:1], (Bblk, D), (0, 1))
    p1 = lax.broadcast_in_dim(pol[:, 1:2], (Bblk, D), (0, 1))
    shift = p0 >= p1
    input_is_empty = (slb - clb) == -1.0
    stack_is_empty = skb <= 1.0
    shift = (shift & (~input_is_empty)) | stack_is_empty
    ski = skb.astype(jnp.int32)
    L1 = jnp.int32(L - 1)
    r1w = jnp.where(shift, jnp.minimum(ski, L1), ski - 2)
    r2w = jnp.where(shift, jnp.minimum(ski, L1),
                    jnp.where(skb >= 3.0, ski - 3, ski - 2))
    r1v[...] = r1w[:, :1]
    r2v[...] = r2w[:, :1]
    v1s[...] = jnp.where(shift, ic, 0.0)
    v2s[...] = jnp.where(shift, ic, jnp.where(skb >= 3.0, red, 0.0))
    out_ref[...] = stack_ref[...]

    def body(b, carry):
        rr1 = r1v[b, 0]
        rr2 = r2v[b, 0]
        out_ref[b, pl.ds(rr1, 1), :] = v1s[pl.ds(b, 1), :]
        out_ref[b, pl.ds(rr2, 1), :] = v2s[pl.ds(b, 1), :]
        return carry

    lax.fori_loop(0, Bblk, body, 0)


def kernel(stack, data, cursors, stack_mask, mask, W_R, b_R, W_S1, b_S1, W_S2, b_S2):
    B, L, D = stack.shape
    H = D // 2
    RL = W_S1.shape[1]
    sp_i, spp_i, ic_i = pl.pallas_call(
        _idx_body,
        out_shape=[jax.ShapeDtypeStruct((B, 1), jnp.int32)] * 3,
    )(cursors, stack_mask)
    gather = _make_gather(B, D)
    sp_rows, spp_rows, ic_rows = gather(
        stack.reshape(B * L, D), data.reshape(B * L, D),
        sp_i.reshape(B), spp_i.reshape(B), ic_i.reshape(B))
    Bblk = 128
    ws2p = jnp.zeros((RL, 128), jnp.float32).at[:, : W_S2.shape[1]].set(W_S2)
    bs2p = jnp.zeros((1, 128), jnp.float32).at[:, : b_S2.shape[0]].set(b_S2[None, :])
    return pl.pallas_call(
        _main_body,
        grid=(B // Bblk,),
        in_specs=[
            pl.BlockSpec((Bblk, D), lambda i: (i, 0)),
            pl.BlockSpec((Bblk, D), lambda i: (i, 0)),
            pl.BlockSpec((Bblk, D), lambda i: (i, 0)),
            pl.BlockSpec((Bblk, L), lambda i: (i, 0)),
            pl.BlockSpec((Bblk, L), lambda i: (i, 0)),
            pl.BlockSpec((Bblk, L), lambda i: (i, 0)),
            pl.BlockSpec((D, 5 * H), lambda i: (0, 0)),
            pl.BlockSpec((1, 5 * H), lambda i: (0, 0)),
            pl.BlockSpec((3 * H, RL), lambda i: (0, 0)),
            pl.BlockSpec((1, RL), lambda i: (0, 0)),
            pl.BlockSpec((RL, 128), lambda i: (0, 0)),
            pl.BlockSpec((1, 128), lambda i: (0, 0)),
            pl.BlockSpec((Bblk, L, D), lambda i: (i, 0, 0)),
        ],
        out_specs=pl.BlockSpec((Bblk, L, D), lambda i: (i, 0, 0)),
        out_shape=jax.ShapeDtypeStruct((B, L, D), jnp.float32),
        scratch_shapes=[
            pltpu.VMEM((Bblk, 1), jnp.int32),
            pltpu.VMEM((Bblk, 1), jnp.int32),
            pltpu.VMEM((Bblk, D), jnp.float32),
            pltpu.VMEM((Bblk, D), jnp.float32),
        ],
        compiler_params=pltpu.CompilerParams(dimension_semantics=("arbitrary",)),
    )(sp_rows, spp_rows, ic_rows, cursors, stack_mask, mask,
      W_R, b_R[None, :], W_S1, b_S1[None, :], ws2p, bs2p, stack)
